# baseline (speedup 1.0000x reference)
import jax
import jax.numpy as jnp
from jax.experimental import pallas as pl
from jax.experimental.pallas import tpu as pltpu

MAX_NUM_TILES = 4
HIDDEN_SIZE = 1280
NUM_PATCHES = 1025
BLOCK_ROWS = 1024


def _body(ids_ref, gate_ref, table_ref, hid_ref, out_ref):
    i = pl.program_id(0)
    r0 = i * BLOCK_ROWS
    seg0 = r0 // NUM_PATCHES
    nseg = ids_ref.shape[0] * MAX_NUM_TILES
    seg1 = jnp.minimum(seg0 + 1, nseg - 1)
    g = jnp.tanh(gate_ref[0])

    def emb(seg):
        row = ids_ref[seg // MAX_NUM_TILES]
        t = seg % MAX_NUM_TILES
        return table_ref[row, pl.ds(t * HIDDEN_SIZE, HIDDEN_SIZE)]

    e0 = (g * emb(seg0))[None, :]
    e1 = (g * emb(seg1))[None, :]
    boundary = (seg0 + 1) * NUM_PATCHES - r0
    in_seg0 = jax.lax.broadcasted_iota(jnp.int32, (BLOCK_ROWS, 1), 0) < boundary
    out_ref[...] = hid_ref[...] + jnp.where(in_seg0, e0, e1)


def kernel(hidden_state, aspect_ratio_ids, embedding_table, gate):
    batch = hidden_state.shape[0]
    rows = batch * MAX_NUM_TILES * NUM_PATCHES
    ids = aspect_ratio_ids.astype(jnp.int32)
    hid2d = hidden_state.reshape(rows, HIDDEN_SIZE)
    grid = pl.cdiv(rows, BLOCK_ROWS)

    out = pl.pallas_call(
        _body,
        grid=(grid,),
        in_specs=[
            pl.BlockSpec(memory_space=pltpu.SMEM),
            pl.BlockSpec(memory_space=pltpu.SMEM),
            pl.BlockSpec(memory_space=pltpu.VMEM),
            pl.BlockSpec((BLOCK_ROWS, HIDDEN_SIZE), lambda i: (i, 0)),
        ],
        out_specs=pl.BlockSpec((BLOCK_ROWS, HIDDEN_SIZE), lambda i: (i, 0)),
        out_shape=jax.ShapeDtypeStruct(hid2d.shape, hid2d.dtype),
        compiler_params=pltpu.CompilerParams(
            dimension_semantics=("arbitrary",),
        ),
    )(ids, gate, embedding_table, hid2d)
    return out.reshape(hidden_state.shape)


# speedup vs baseline: 3.9268x; 3.9268x over previous
import jax
import jax.numpy as jnp
from jax.experimental import pallas as pl
from jax.experimental.pallas import tpu as pltpu

MAX_NUM_TILES = 4
HIDDEN_SIZE = 1280
NUM_PATCHES = 1025


def _body(ids_ref, gate_ref, table_ref, hid_ref, out_ref):
    b = pl.program_id(0)
    th = pl.program_id(1)
    row = ids_ref[b]
    g = jnp.tanh(gate_ref[0])
    emb = table_ref[row, pl.ds(th * 2, 2)]
    out_ref[...] = hid_ref[...] + (g * emb)[None]


def kernel(hidden_state, aspect_ratio_ids, embedding_table, gate):
    batch = hidden_state.shape[0]
    ids = aspect_ratio_ids.astype(jnp.int32)
    table4d = embedding_table.reshape(
        embedding_table.shape[0], MAX_NUM_TILES, 1, HIDDEN_SIZE)

    return pl.pallas_call(
        _body,
        grid=(batch, MAX_NUM_TILES // 2),
        in_specs=[
            pl.BlockSpec(memory_space=pltpu.SMEM),
            pl.BlockSpec(memory_space=pltpu.SMEM),
            pl.BlockSpec(memory_space=pltpu.VMEM),
            pl.BlockSpec((1, 2, NUM_PATCHES, HIDDEN_SIZE),
                         lambda b, th: (b, th, 0, 0)),
        ],
        out_specs=pl.BlockSpec((1, 2, NUM_PATCHES, HIDDEN_SIZE),
                               lambda b, th: (b, th, 0, 0)),
        out_shape=jax.ShapeDtypeStruct(hidden_state.shape, hidden_state.dtype),
        compiler_params=pltpu.CompilerParams(
            dimension_semantics=("arbitrary", "arbitrary"),
            vmem_limit_bytes=63 * 1024 * 1024,
        ),
    )(ids, gate, table4d, hidden_state)
